# Initial kernel scaffold; baseline (speedup 1.0000x reference)
import jax
import jax.numpy as jnp
from jax.experimental import pallas as pl


def kernel(iobs, cloud_fraction_delta):
    raise NotImplementedError("write your pallas kernel here")


# speedup vs baseline: 139.3447x; 139.3447x over previous
import functools

import jax
import jax.numpy as jnp
from jax import lax
from jax.experimental import pallas as pl
from jax.experimental.pallas import tpu as pltpu
from jax.experimental.pallas import tpu_sc as plsc

_N = 16384 * 200
_NC = 2
_NS = 16
_NW = _NC * _NS
_PER_W = _N // _NW
_CHUNK = 25600
_NCHUNK = _PER_W // _CHUNK

_mesh = plsc.VectorSubcoreMesh(core_axis_name="c", subcore_axis_name="s")


@functools.partial(
    pl.kernel,
    out_type=jax.ShapeDtypeStruct((_N,), jnp.float32),
    mesh=_mesh,
    scratch_types=[
        pltpu.VMEM((_CHUNK,), jnp.int32),
        pltpu.VMEM((_CHUNK,), jnp.float32),
        pltpu.SemaphoreType.DMA,
    ],
)
def _gather_kernel(idx_hbm, table_hbm, out_hbm, idx_v, out_v, sem):
    wid = lax.axis_index("s") * _NC + lax.axis_index("c")
    base = wid * _PER_W
    for i in range(_NCHUNK):
        off = base + i * _CHUNK
        pltpu.sync_copy(idx_hbm.at[pl.ds(off, _CHUNK)], idx_v)
        pltpu.async_copy(table_hbm.at[idx_v], out_v, sem).wait()
        pltpu.sync_copy(out_v, out_hbm.at[pl.ds(off, _CHUNK)])


def kernel(iobs, cloud_fraction_delta):
    flat = iobs.reshape(-1)
    out = _gather_kernel(flat, cloud_fraction_delta)
    return out.reshape(iobs.shape)


# speedup vs baseline: 216.5391x; 1.5540x over previous
import functools

import jax
import jax.numpy as jnp
from jax import lax
from jax.experimental import pallas as pl
from jax.experimental.pallas import tpu as pltpu
from jax.experimental.pallas import tpu_sc as plsc

_NOBS = 1000000
_N = 16384 * 200
_NC = 2
_NS = 16
_NW = _NC * _NS
_PER_W = _N // _NW
_CHUNK = 25600
_NCHUNK = _PER_W // _CHUNK
_STAGE = 62496
_SCH = 20832

_mesh = plsc.VectorSubcoreMesh(core_axis_name="c", subcore_axis_name="s")


@functools.partial(
    pl.kernel,
    out_type=jax.ShapeDtypeStruct((_N,), jnp.float32),
    mesh=_mesh,
    scratch_types=[
        pltpu.VMEM_SHARED((_NOBS,), jnp.float32),
        pltpu.VMEM((_CHUNK,), jnp.int32),
        pltpu.VMEM((_CHUNK,), jnp.float32),
        pltpu.SemaphoreType.DMA,
    ],
)
def _gather_kernel(idx_hbm, table_hbm, out_hbm, tbl_s, idx_v, out_v, sem):
    s = lax.axis_index("s")
    wid = s * _NC + lax.axis_index("c")

    for j in range(_STAGE // _SCH):
        soff = s * _STAGE + j * _SCH
        pltpu.sync_copy(table_hbm.at[pl.ds(soff, _SCH)], out_v.at[pl.ds(0, _SCH)])
        pltpu.sync_copy(out_v.at[pl.ds(0, _SCH)], tbl_s.at[pl.ds(soff, _SCH)])

    @pl.when(s == 0)
    def _():
        rem = _NOBS - _NS * _STAGE
        roff = _NS * _STAGE
        pltpu.sync_copy(table_hbm.at[pl.ds(roff, rem)], out_v.at[pl.ds(0, rem)])
        pltpu.sync_copy(out_v.at[pl.ds(0, rem)], tbl_s.at[pl.ds(roff, rem)])

    plsc.subcore_barrier()

    base = wid * _PER_W
    for i in range(_NCHUNK):
        off = base + i * _CHUNK
        pltpu.sync_copy(idx_hbm.at[pl.ds(off, _CHUNK)], idx_v)
        pltpu.async_copy(tbl_s.at[idx_v], out_v, sem).wait()
        pltpu.sync_copy(out_v, out_hbm.at[pl.ds(off, _CHUNK)])


def kernel(iobs, cloud_fraction_delta):
    flat = iobs.reshape(-1)
    out = _gather_kernel(flat, cloud_fraction_delta)
    return out.reshape(iobs.shape)


# speedup vs baseline: 228.2955x; 1.0543x over previous
import functools

import jax
import jax.numpy as jnp
from jax import lax
from jax.experimental import pallas as pl
from jax.experimental.pallas import tpu as pltpu
from jax.experimental.pallas import tpu_sc as plsc

_NOBS = 1000000
_N = 16384 * 200
_NC = 2
_NS = 16
_NW = _NC * _NS
_PER_W = _N // _NW
_CHUNK = 12800
_NCHUNK = _PER_W // _CHUNK
_STAGE = 62496
_SCH = 10416

_mesh = plsc.VectorSubcoreMesh(core_axis_name="c", subcore_axis_name="s")


@functools.partial(
    pl.kernel,
    out_type=jax.ShapeDtypeStruct((_N,), jnp.float32),
    mesh=_mesh,
    scratch_types=[
        pltpu.VMEM_SHARED((_NOBS,), jnp.float32),
        pltpu.VMEM((_CHUNK,), jnp.int32),
        pltpu.VMEM((_CHUNK,), jnp.int32),
        pltpu.VMEM((_CHUNK,), jnp.float32),
        pltpu.VMEM((_CHUNK,), jnp.float32),
        pltpu.VMEM((_SCH,), jnp.float32),
        pltpu.SemaphoreType.DMA((2,)),
        pltpu.SemaphoreType.DMA((2,)),
        pltpu.SemaphoreType.DMA((2,)),
    ],
)
def _gather_kernel(idx_hbm, table_hbm, out_hbm, tbl_s, idx_v0, idx_v1,
                   out_v0, out_v1, stg_v, lsem, gsem, ssem):
    idx_v = [idx_v0, idx_v1]
    out_v = [out_v0, out_v1]
    s = lax.axis_index("s")
    wid = s * _NC + lax.axis_index("c")
    base = wid * _PER_W

    pltpu.async_copy(idx_hbm.at[pl.ds(base, _CHUNK)], idx_v[0], lsem.at[0])

    for j in range(_STAGE // _SCH):
        soff = s * _STAGE + j * _SCH
        pltpu.sync_copy(table_hbm.at[pl.ds(soff, _SCH)], stg_v)
        pltpu.sync_copy(stg_v, tbl_s.at[pl.ds(soff, _SCH)])

    @pl.when(s == 0)
    def _():
        rem = _NOBS - _NS * _STAGE
        roff = _NS * _STAGE
        bv = stg_v.at[pl.ds(0, rem)]
        pltpu.sync_copy(table_hbm.at[pl.ds(roff, rem)], bv)
        pltpu.sync_copy(bv, tbl_s.at[pl.ds(roff, rem)])

    plsc.subcore_barrier()

    for i in range(_NCHUNK):
        b = i % 2
        pltpu.make_async_copy(
            idx_hbm.at[pl.ds(base + i * _CHUNK, _CHUNK)], idx_v[b],
            lsem.at[b]).wait()
        if i >= 2:
            pltpu.make_async_copy(
                out_v[b], out_hbm.at[pl.ds(base + (i - 2) * _CHUNK, _CHUNK)],
                ssem.at[b]).wait()
        pltpu.async_copy(tbl_s.at[idx_v[b]], out_v[b], gsem.at[b])
        if i >= 1:
            pltpu.make_async_copy(
                tbl_s.at[idx_v[1 - b]], out_v[1 - b],
                gsem.at[1 - b]).wait()
            pltpu.async_copy(
                out_v[1 - b],
                out_hbm.at[pl.ds(base + (i - 1) * _CHUNK, _CHUNK)],
                ssem.at[1 - b])
        if i + 1 < _NCHUNK:
            pltpu.async_copy(
                idx_hbm.at[pl.ds(base + (i + 1) * _CHUNK, _CHUNK)],
                idx_v[1 - b], lsem.at[1 - b])

    last = (_NCHUNK - 1) % 2
    pltpu.make_async_copy(
        tbl_s.at[idx_v[last]], out_v[last], gsem.at[last]).wait()
    pltpu.async_copy(
        out_v[last],
        out_hbm.at[pl.ds(base + (_NCHUNK - 1) * _CHUNK, _CHUNK)],
        ssem.at[last])
    pltpu.make_async_copy(
        out_v[1 - last],
        out_hbm.at[pl.ds(base + (_NCHUNK - 2) * _CHUNK, _CHUNK)],
        ssem.at[1 - last]).wait()
    pltpu.make_async_copy(
        out_v[last],
        out_hbm.at[pl.ds(base + (_NCHUNK - 1) * _CHUNK, _CHUNK)],
        ssem.at[last]).wait()


def kernel(iobs, cloud_fraction_delta):
    flat = iobs.reshape(-1)
    out = _gather_kernel(flat, cloud_fraction_delta)
    return out.reshape(iobs.shape)
